# baseline (speedup 1.0000x reference)
import functools

import jax
import jax.numpy as jnp
from jax import lax
from jax.experimental import pallas as pl
from jax.experimental.pallas import tpu as pltpu
from jax.experimental.pallas import tpu_sc as plsc

_B = 16384
_D = 64
_VD = 512
_NI = 100000
_L2_LAMBDA = 1e-05

_ROWS_PER_BLK = 1000
_CHUNK = 128
_LOSS_BLK = 2048


def _proj_body(ivf_ref, wt_ref, out_ref):
    out_ref[...] = jnp.dot(ivf_ref[...], wt_ref[...],
                           preferred_element_type=jnp.float32)


def _project_items(ivf, w_t):
    grid = _NI // _ROWS_PER_BLK
    return pl.pallas_call(
        _proj_body,
        grid=(grid,),
        in_specs=[
            pl.BlockSpec((_ROWS_PER_BLK, _VD), lambda i: (i, 0)),
            pl.BlockSpec((_VD, _D), lambda i: (0, 0)),
        ],
        out_specs=pl.BlockSpec((_ROWS_PER_BLK, _D), lambda i: (i, 0)),
        out_shape=jax.ShapeDtypeStruct((_NI, _D), jnp.float32),
    )(ivf, w_t)


def _loss_body(ue, pe, ne, uv, pve, nve, out_ref):
    i = pl.program_id(0)
    ue_ = ue[...]
    pe_ = pe[...]
    ne_ = ne[...]
    uv_ = uv[...]
    pos = jnp.sum(ue_ * pe_ + uv_ * pve[...], axis=1)
    neg = jnp.sum(ue_ * ne_ + uv_ * nve[...], axis=1)
    x = pos - neg
    cf = jnp.maximum(-x, 0.0) + jnp.log1p(jnp.exp(-jnp.abs(x)))
    l2 = 0.5 * jnp.sum(ue_ * ue_ + pe_ * pe_ + ne_ * ne_, axis=1)
    part = jnp.sum(cf + _L2_LAMBDA * l2)

    @pl.when(i == 0)
    def _():
        out_ref[0, 0] = 0.0

    out_ref[0, 0] += part

    @pl.when(i == pl.num_programs(0) - 1)
    def _():
        out_ref[0, 0] = out_ref[0, 0] / float(_B)


def _fused_loss(ue, pe, ne, uv, pve, nve):
    grid = _B // _LOSS_BLK
    blk = pl.BlockSpec((_LOSS_BLK, _D), lambda i: (i, 0))
    return pl.pallas_call(
        _loss_body,
        grid=(grid,),
        in_specs=[blk] * 6,
        out_specs=pl.BlockSpec((1, 1), lambda i: (0, 0),
                               memory_space=pltpu.SMEM),
        out_shape=jax.ShapeDtypeStruct((1, 1), jnp.float32),
    )(ue, pe, ne, uv, pve, nve)


def _gather_kernel_body(uid, pid, nid, ue_t, ie_t, uv_t, pv_t,
                        o_ue, o_pe, o_ne, o_uv, o_pve, o_nve,
                        idx_v, rows_v, sem):
    nc = 2
    wid = lax.axis_index("s") * nc + lax.axis_index("c")
    per_w = _B // 32
    base0 = wid * per_w

    def gather_one(table, out, base):
        pltpu.async_copy(table.at[idx_v], rows_v, sem).wait()
        pltpu.sync_copy(rows_v, out.at[pl.ds(base, _CHUNK)])

    for c in range(per_w // _CHUNK):
        base = base0 + c * _CHUNK
        pltpu.sync_copy(uid.at[pl.ds(base, _CHUNK)], idx_v)
        gather_one(ue_t, o_ue, base)
        gather_one(uv_t, o_uv, base)
        pltpu.sync_copy(pid.at[pl.ds(base, _CHUNK)], idx_v)
        gather_one(ie_t, o_pe, base)
        gather_one(pv_t, o_pve, base)
        pltpu.sync_copy(nid.at[pl.ds(base, _CHUNK)], idx_v)
        gather_one(ie_t, o_ne, base)
        gather_one(pv_t, o_nve, base)


def _gather_all(uid, pid, nid, ue_t, ie_t, uv_t, pv_t):
    mesh = plsc.VectorSubcoreMesh(core_axis_name="c", subcore_axis_name="s",
                                  num_cores=2, num_subcores=16)
    rows = jax.ShapeDtypeStruct((_B, _D), jnp.float32)
    k = pl.kernel(
        _gather_kernel_body,
        out_type=(rows,) * 6,
        mesh=mesh,
        scratch_types=[
            pltpu.VMEM((_CHUNK,), jnp.int32),
            pltpu.VMEM((_CHUNK, _D), jnp.float32),
            pltpu.SemaphoreType.DMA,
        ],
        compiler_params=pltpu.CompilerParams(use_tc_tiling_on_sc=False),
    )
    return k(uid, pid, nid, ue_t, ie_t, uv_t, pv_t)


def kernel(user_ids, item_pos_ids, item_neg_ids, user_embed, item_embed,
           user_visual_embed, item_visual_feature, W_vis):
    proj = _project_items(item_visual_feature, W_vis.T)
    ue, pe, ne, uv, pve, nve = _gather_all(
        user_ids, item_pos_ids, item_neg_ids,
        user_embed, item_embed, user_visual_embed, proj)
    loss = _fused_loss(ue, pe, ne, uv, pve, nve)
    return loss[0, 0]


# speedup vs baseline: 1.0483x; 1.0483x over previous
import functools

import jax
import jax.numpy as jnp
from jax import lax
from jax.experimental import pallas as pl
from jax.experimental.pallas import tpu as pltpu
from jax.experimental.pallas import tpu_sc as plsc

_B = 16384
_D = 64
_VD = 512
_NI = 100000
_NU = 1000000
_L2_LAMBDA = 1e-05

_ROWS_PER_BLK = 1000
_CHUNK = 128
_NW = 32
_PER_W = _B // _NW
_LOSS_BLK = 2048


def _fuse_items_body(ie_ref, ivf_ref, wt_ref, out_ref):
    out_ref[:, :_D] = ie_ref[...]
    out_ref[:, _D:] = jnp.dot(ivf_ref[...], wt_ref[...],
                              preferred_element_type=jnp.float32)


def _fuse_items(ie, ivf, w_t):
    grid = _NI // _ROWS_PER_BLK
    return pl.pallas_call(
        _fuse_items_body,
        grid=(grid,),
        in_specs=[
            pl.BlockSpec((_ROWS_PER_BLK, _D), lambda i: (i, 0)),
            pl.BlockSpec((_ROWS_PER_BLK, _VD), lambda i: (i, 0)),
            pl.BlockSpec((_VD, _D), lambda i: (0, 0)),
        ],
        out_specs=pl.BlockSpec((_ROWS_PER_BLK, 2 * _D), lambda i: (i, 0)),
        out_shape=jax.ShapeDtypeStruct((_NI, 2 * _D), jnp.float32),
    )(ie, ivf, w_t)


def _gather_kernel_body(uid, pid, nid, uep_t, uvp_t, it_t,
                        o_uep, o_uvp, o_it_p, o_it_n,
                        idx_v, rows_v, sem):
    wid = lax.axis_index("s") * 2 + lax.axis_index("c")
    base0 = wid * _PER_W

    def gather_one(table, out, base):
        pltpu.async_copy(table.at[idx_v], rows_v, sem).wait()
        pltpu.sync_copy(rows_v, out.at[pl.ds(base, _CHUNK)])

    for c in range(_PER_W // _CHUNK):
        base = base0 + c * _CHUNK
        pltpu.sync_copy(uid.at[pl.ds(base, _CHUNK)], idx_v)
        for g in range(_CHUNK // 16):
            sl = pl.ds(g * 16, 16)
            idx_v[sl] = lax.shift_right_logical(idx_v[sl], 1)
        gather_one(uep_t, o_uep, base)
        gather_one(uvp_t, o_uvp, base)
        pltpu.sync_copy(pid.at[pl.ds(base, _CHUNK)], idx_v)
        gather_one(it_t, o_it_p, base)
        pltpu.sync_copy(nid.at[pl.ds(base, _CHUNK)], idx_v)
        gather_one(it_t, o_it_n, base)


def _gather_all(uid, pid, nid, uep_t, uvp_t, it_t):
    mesh = plsc.VectorSubcoreMesh(core_axis_name="c", subcore_axis_name="s",
                                  num_cores=2, num_subcores=16)
    rows = jax.ShapeDtypeStruct((_B, 2 * _D), jnp.float32)
    k = pl.kernel(
        _gather_kernel_body,
        out_type=(rows,) * 4,
        mesh=mesh,
        scratch_types=[
            pltpu.VMEM((_CHUNK,), jnp.int32),
            pltpu.VMEM((_CHUNK, 2 * _D), jnp.float32),
            pltpu.SemaphoreType.DMA,
        ],
    )
    return k(uid, pid, nid, uep_t, uvp_t, it_t)


def _loss_body(uid, uep, uvp, itp, itn, out_ref):
    i = pl.program_id(0)
    half = (uid[...] & 1) == 1
    uep_ = uep[...]
    uvp_ = uvp[...]
    ue = jnp.where(half, uep_[:, _D:], uep_[:, :_D])
    uv = jnp.where(half, uvp_[:, _D:], uvp_[:, :_D])
    ucomb = jnp.concatenate([ue, uv], axis=1)
    itp_ = itp[...]
    itn_ = itn[...]
    pos = jnp.sum(ucomb * itp_, axis=1)
    neg = jnp.sum(ucomb * itn_, axis=1)
    x = pos - neg
    cf = jnp.maximum(-x, 0.0) + jnp.log1p(jnp.exp(-jnp.abs(x)))
    l2 = 0.5 * (jnp.sum(ue * ue, axis=1)
                + jnp.sum(itp_[:, :_D] * itp_[:, :_D], axis=1)
                + jnp.sum(itn_[:, :_D] * itn_[:, :_D], axis=1))
    part = jnp.sum(cf + _L2_LAMBDA * l2)

    @pl.when(i == 0)
    def _():
        out_ref[0, 0] = 0.0

    out_ref[0, 0] += part

    @pl.when(i == pl.num_programs(0) - 1)
    def _():
        out_ref[0, 0] = out_ref[0, 0] / float(_B)


def _fused_loss(uid2, uep, uvp, itp, itn):
    grid = _B // _LOSS_BLK
    blk = pl.BlockSpec((_LOSS_BLK, 2 * _D), lambda i: (i, 0))
    return pl.pallas_call(
        _loss_body,
        grid=(grid,),
        in_specs=[pl.BlockSpec((_LOSS_BLK, 1), lambda i: (i, 0))] + [blk] * 4,
        out_specs=pl.BlockSpec((1, 1), lambda i: (0, 0),
                               memory_space=pltpu.SMEM),
        out_shape=jax.ShapeDtypeStruct((1, 1), jnp.float32),
    )(uid2, uep, uvp, itp, itn)


def kernel(user_ids, item_pos_ids, item_neg_ids, user_embed, item_embed,
           user_visual_embed, item_visual_feature, W_vis):
    fused_items = _fuse_items(item_embed, item_visual_feature, W_vis.T)
    uep_t = user_embed.reshape(_NU // 2, 2 * _D)
    uvp_t = user_visual_embed.reshape(_NU // 2, 2 * _D)
    uep, uvp, itp, itn = _gather_all(
        user_ids, item_pos_ids, item_neg_ids, uep_t, uvp_t, fused_items)
    loss = _fused_loss(user_ids.reshape(_B, 1), uep, uvp, itp, itn)
    return loss[0, 0]


# speedup vs baseline: 1.3995x; 1.3350x over previous
import functools

import jax
import jax.numpy as jnp
from jax import lax
from jax.experimental import pallas as pl
from jax.experimental.pallas import tpu as pltpu
from jax.experimental.pallas import tpu_sc as plsc

_B = 16384
_D = 64
_VD = 512
_NI = 100000
_NU = 1000000
_L2_LAMBDA = 1e-05

_ROWS_PER_BLK = 1000
_CHUNK = 128
_NW = 32
_PER_W = _B // _NW
_LOSS_BLK = 2048


def _fuse_items_body(ie_ref, ivf_ref, wt_ref, out_ref):
    out_ref[:, :_D] = ie_ref[...]
    out_ref[:, _D:] = jnp.dot(ivf_ref[...], wt_ref[...],
                              preferred_element_type=jnp.float32)


def _fuse_items(ie, ivf, w_t):
    grid = _NI // _ROWS_PER_BLK
    return pl.pallas_call(
        _fuse_items_body,
        grid=(grid,),
        in_specs=[
            pl.BlockSpec((_ROWS_PER_BLK, _D), lambda i: (i, 0)),
            pl.BlockSpec((_ROWS_PER_BLK, _VD), lambda i: (i, 0)),
            pl.BlockSpec((_VD, _D), lambda i: (0, 0)),
        ],
        out_specs=pl.BlockSpec((_ROWS_PER_BLK, 2 * _D), lambda i: (i, 0)),
        out_shape=jax.ShapeDtypeStruct((_NI, 2 * _D), jnp.float32),
    )(ie, ivf, w_t)


def _gather_kernel_body(uid, pid, nid, ue2, uv2, it_t,
                        o_ue, o_uv, o_it_p, o_it_n,
                        uidv, idx_v, rows_v, uerows, uvrows, sem, semi):
    wid = lax.axis_index("s") * 2 + lax.axis_index("c")
    base0 = wid * _PER_W
    lanes = lax.iota(jnp.int32, 16)

    for c in range(_PER_W // _CHUNK):
        base = base0 + c * _CHUNK

        pltpu.sync_copy(uid.at[pl.ds(base, _CHUNK)], uidv)

        @pl.loop(0, _CHUNK // 16)
        def _(g):
            vec = uidv[pl.ds(g * 16, 16)]
            descs = []
            for j in range(16):
                rid = jnp.sum(jnp.where(lanes == j, vec, 0))
                e = g * 16 + j
                descs.append(pltpu.async_copy(
                    ue2.at[pl.ds(rid, 1)], uerows.at[pl.ds(e, 1)], sem))
                descs.append(pltpu.async_copy(
                    uv2.at[pl.ds(rid, 1)], uvrows.at[pl.ds(e, 1)], sem))
            for dsc in descs:
                dsc.wait()

        pltpu.sync_copy(uerows, o_ue.at[pl.ds(base, _CHUNK)])
        pltpu.sync_copy(uvrows, o_uv.at[pl.ds(base, _CHUNK)])

        pltpu.sync_copy(pid.at[pl.ds(base, _CHUNK)], idx_v)
        pltpu.async_copy(it_t.at[idx_v], rows_v, semi).wait()
        pltpu.sync_copy(rows_v, o_it_p.at[pl.ds(base, _CHUNK)])
        pltpu.sync_copy(nid.at[pl.ds(base, _CHUNK)], idx_v)
        pltpu.async_copy(it_t.at[idx_v], rows_v, semi).wait()
        pltpu.sync_copy(rows_v, o_it_n.at[pl.ds(base, _CHUNK)])


def _gather_all(uid, pid, nid, ue2, uv2, it_t):
    mesh = plsc.VectorSubcoreMesh(core_axis_name="c", subcore_axis_name="s",
                                  num_cores=2, num_subcores=16)
    urows = jax.ShapeDtypeStruct((_B, _D), jnp.float32)
    irows = jax.ShapeDtypeStruct((_B, 2 * _D), jnp.float32)
    k = pl.kernel(
        _gather_kernel_body,
        out_type=(urows, urows, irows, irows),
        mesh=mesh,
        scratch_types=[
            pltpu.VMEM((_CHUNK,), jnp.int32),
            pltpu.VMEM((_CHUNK,), jnp.int32),
            pltpu.VMEM((_CHUNK, 2 * _D), jnp.float32),
            pltpu.VMEM((_CHUNK, _D), jnp.float32),
            pltpu.VMEM((_CHUNK, _D), jnp.float32),
            pltpu.SemaphoreType.DMA,
            pltpu.SemaphoreType.DMA,
        ],
        compiler_params=pltpu.CompilerParams(needs_layout_passes=False),
    )
    return k(uid, pid, nid, ue2, uv2, it_t)


def _loss_body(ue, uv, itp, itn, out_ref):
    i = pl.program_id(0)
    ue_ = ue[...]
    uv_ = uv[...]
    ucomb = jnp.concatenate([ue_, uv_], axis=1)
    itp_ = itp[...]
    itn_ = itn[...]
    pos = jnp.sum(ucomb * itp_, axis=1)
    neg = jnp.sum(ucomb * itn_, axis=1)
    x = pos - neg
    cf = jnp.maximum(-x, 0.0) + jnp.log1p(jnp.exp(-jnp.abs(x)))
    l2 = 0.5 * (jnp.sum(ue_ * ue_, axis=1)
                + jnp.sum(itp_[:, :_D] * itp_[:, :_D], axis=1)
                + jnp.sum(itn_[:, :_D] * itn_[:, :_D], axis=1))
    part = jnp.sum(cf + _L2_LAMBDA * l2)

    @pl.when(i == 0)
    def _():
        out_ref[0, 0] = 0.0

    out_ref[0, 0] += part

    @pl.when(i == pl.num_programs(0) - 1)
    def _():
        out_ref[0, 0] = out_ref[0, 0] / float(_B)


def _fused_loss(ue, uv, itp, itn):
    grid = _B // _LOSS_BLK
    ublk = pl.BlockSpec((_LOSS_BLK, _D), lambda i: (i, 0))
    iblk = pl.BlockSpec((_LOSS_BLK, 2 * _D), lambda i: (i, 0))
    return pl.pallas_call(
        _loss_body,
        grid=(grid,),
        in_specs=[ublk, ublk, iblk, iblk],
        out_specs=pl.BlockSpec((1, 1), lambda i: (0, 0),
                               memory_space=pltpu.SMEM),
        out_shape=jax.ShapeDtypeStruct((1, 1), jnp.float32),
    )(ue, uv, itp, itn)


def kernel(user_ids, item_pos_ids, item_neg_ids, user_embed, item_embed,
           user_visual_embed, item_visual_feature, W_vis):
    fused_items = _fuse_items(item_embed, item_visual_feature, W_vis.T)
    ue, uv, itp, itn = _gather_all(
        user_ids, item_pos_ids, item_neg_ids,
        user_embed, user_visual_embed, fused_items)
    loss = _fused_loss(ue, uv, itp, itn)
    return loss[0, 0]


# speedup vs baseline: 1.4064x; 1.0049x over previous
import functools

import jax
import jax.numpy as jnp
from jax import lax
from jax.experimental import pallas as pl
from jax.experimental.pallas import tpu as pltpu
from jax.experimental.pallas import tpu_sc as plsc

_B = 16384
_D = 64
_VD = 512
_NI = 100000
_NU = 1000000
_L2_LAMBDA = 1e-05

_ROWS_PER_BLK = 1000
_CHUNK = 128
_NW = 32
_PER_W = _B // _NW
_LOSS_BLK = 2048


def _fuse_items_body(ie_ref, ivf_ref, wt_ref, out_ref):
    out_ref[:, :_D] = ie_ref[...]
    out_ref[:, _D:] = jnp.dot(ivf_ref[...], wt_ref[...],
                              preferred_element_type=jnp.float32)


def _fuse_items(ie, ivf, w_t):
    grid = _NI // _ROWS_PER_BLK
    return pl.pallas_call(
        _fuse_items_body,
        grid=(grid,),
        in_specs=[
            pl.BlockSpec((_ROWS_PER_BLK, _D), lambda i: (i, 0)),
            pl.BlockSpec((_ROWS_PER_BLK, _VD), lambda i: (i, 0)),
            pl.BlockSpec((_VD, _D), lambda i: (0, 0)),
        ],
        out_specs=pl.BlockSpec((_ROWS_PER_BLK, 2 * _D), lambda i: (i, 0)),
        out_shape=jax.ShapeDtypeStruct((_NI, 2 * _D), jnp.float32),
    )(ie, ivf, w_t)


def _gather_kernel_body(uid, pid, nid, ue2, uv2, it_t,
                        o_ue, o_uv, o_it_p, o_it_n,
                        uidv, idx_v, rows_v, uerows, uvrows, sem, semi):
    wid = lax.axis_index("s") * 2 + lax.axis_index("c")
    base0 = wid * _PER_W

    for c in range(_PER_W // _CHUNK):
        base = base0 + c * _CHUNK

        pltpu.sync_copy(uid.at[pl.ds(base, _CHUNK)], uidv)

        @pl.loop(0, _CHUNK // 16)
        def _(g):
            vec = uidv[pl.ds(g * 16, 16)]
            descs = []
            for j in range(16):
                e = g * 16 + j
                rid = vec[j]
                descs.append(pltpu.async_copy(
                    ue2.at[pl.ds(rid, 1)], uerows.at[pl.ds(e, 1)], sem))
                descs.append(pltpu.async_copy(
                    uv2.at[pl.ds(rid, 1)], uvrows.at[pl.ds(e, 1)], sem))
            for dsc in descs:
                dsc.wait()

        pltpu.sync_copy(uerows, o_ue.at[pl.ds(base, _CHUNK)])
        pltpu.sync_copy(uvrows, o_uv.at[pl.ds(base, _CHUNK)])

        pltpu.sync_copy(pid.at[pl.ds(base, _CHUNK)], idx_v)
        pltpu.async_copy(it_t.at[idx_v], rows_v, semi).wait()
        pltpu.sync_copy(rows_v, o_it_p.at[pl.ds(base, _CHUNK)])
        pltpu.sync_copy(nid.at[pl.ds(base, _CHUNK)], idx_v)
        pltpu.async_copy(it_t.at[idx_v], rows_v, semi).wait()
        pltpu.sync_copy(rows_v, o_it_n.at[pl.ds(base, _CHUNK)])


def _gather_all(uid, pid, nid, ue2, uv2, it_t):
    mesh = plsc.VectorSubcoreMesh(core_axis_name="c", subcore_axis_name="s",
                                  num_cores=2, num_subcores=16)
    urows = jax.ShapeDtypeStruct((_B, _D), jnp.float32)
    irows = jax.ShapeDtypeStruct((_B, 2 * _D), jnp.float32)
    k = pl.kernel(
        _gather_kernel_body,
        out_type=(urows, urows, irows, irows),
        mesh=mesh,
        scratch_types=[
            pltpu.VMEM((_CHUNK,), jnp.int32),
            pltpu.VMEM((_CHUNK,), jnp.int32),
            pltpu.VMEM((_CHUNK, 2 * _D), jnp.float32),
            pltpu.VMEM((_CHUNK, _D), jnp.float32),
            pltpu.VMEM((_CHUNK, _D), jnp.float32),
            pltpu.SemaphoreType.DMA,
            pltpu.SemaphoreType.DMA,
        ],
    )
    return k(uid, pid, nid, ue2, uv2, it_t)


def _loss_body(ue, uv, itp, itn, out_ref):
    i = pl.program_id(0)
    ue_ = ue[...]
    uv_ = uv[...]
    ucomb = jnp.concatenate([ue_, uv_], axis=1)
    itp_ = itp[...]
    itn_ = itn[...]
    pos = jnp.sum(ucomb * itp_, axis=1)
    neg = jnp.sum(ucomb * itn_, axis=1)
    x = pos - neg
    cf = jnp.maximum(-x, 0.0) + jnp.log1p(jnp.exp(-jnp.abs(x)))
    l2 = 0.5 * (jnp.sum(ue_ * ue_, axis=1)
                + jnp.sum(itp_[:, :_D] * itp_[:, :_D], axis=1)
                + jnp.sum(itn_[:, :_D] * itn_[:, :_D], axis=1))
    part = jnp.sum(cf + _L2_LAMBDA * l2)

    @pl.when(i == 0)
    def _():
        out_ref[0, 0] = 0.0

    out_ref[0, 0] += part

    @pl.when(i == pl.num_programs(0) - 1)
    def _():
        out_ref[0, 0] = out_ref[0, 0] / float(_B)


def _fused_loss(ue, uv, itp, itn):
    grid = _B // _LOSS_BLK
    ublk = pl.BlockSpec((_LOSS_BLK, _D), lambda i: (i, 0))
    iblk = pl.BlockSpec((_LOSS_BLK, 2 * _D), lambda i: (i, 0))
    return pl.pallas_call(
        _loss_body,
        grid=(grid,),
        in_specs=[ublk, ublk, iblk, iblk],
        out_specs=pl.BlockSpec((1, 1), lambda i: (0, 0),
                               memory_space=pltpu.SMEM),
        out_shape=jax.ShapeDtypeStruct((1, 1), jnp.float32),
    )(ue, uv, itp, itn)


def kernel(user_ids, item_pos_ids, item_neg_ids, user_embed, item_embed,
           user_visual_embed, item_visual_feature, W_vis):
    fused_items = _fuse_items(item_embed, item_visual_feature, W_vis.T)
    ue, uv, itp, itn = _gather_all(
        user_ids, item_pos_ids, item_neg_ids,
        user_embed, user_visual_embed, fused_items)
    loss = _fused_loss(ue, uv, itp, itn)
    return loss[0, 0]
